# baseline (speedup 1.0000x reference)
import functools

import jax
import jax.numpy as jnp
from jax import lax
from jax.experimental import pallas as pl
from jax.experimental.pallas import tpu as pltpu

T = 2048
H = 1024
E = 8
K = 2
I = 512
BM = 1024
MC = 512
NSL = E + 1
NI = T // BM


def _act_body(x_ref, rwt_ref, wgu_ref, wgun_ref, sgu_ref, wdslice_ref,
              sd_ref, act_ref, wdbf_ref, wbf_ref, xb_ref, comb_ref):
    e = pl.program_id(0)
    par = jnp.remainder(e, 2)
    nxt = jnp.remainder(e + 1, 2)

    @pl.when(e == 0)
    def _():
        wbf_ref[0] = wgu_ref[0].astype(jnp.bfloat16)

    @pl.when(e < E - 1)
    def _():
        wbf_ref[nxt] = wgun_ref[0].astype(jnp.bfloat16)

    @pl.when(e == E - 1)
    def _():
        wbf_ref[nxt] = sgu_ref[...].astype(jnp.bfloat16)

    @pl.when(e < E)
    def _():
        wdbf_ref[...] = wdslice_ref[0].astype(jnp.bfloat16)

    @pl.when(e == E)
    def _():
        wdbf_ref[...] = sd_ref[...].astype(jnp.bfloat16)

    @pl.when(e == 0)
    def _():
        x = x_ref[...]
        xb_ref[...] = x.astype(jnp.bfloat16)
        logits = lax.dot_general(
            x, rwt_ref[...], (((1,), (0,)), ((), ())),
            precision=lax.Precision.DEFAULT,
            preferred_element_type=jnp.float32)
        m = jnp.max(logits, axis=-1, keepdims=True)
        ex = jnp.exp(logits - m)
        probs = ex / jnp.sum(ex, axis=-1, keepdims=True)
        lane = lax.broadcasted_iota(jnp.int32, probs.shape, 1)
        p1 = jnp.max(probs, axis=-1, keepdims=True)
        i1 = jnp.min(jnp.where(probs == p1, lane, E), axis=-1, keepdims=True)
        mask1 = lane == i1
        rest = jnp.where(mask1, -jnp.inf, probs)
        p2 = jnp.max(rest, axis=-1, keepdims=True)
        i2 = jnp.min(jnp.where(rest == p2, lane, E), axis=-1, keepdims=True)
        mask2 = lane == i2
        denom = p1 + p2
        comb_ref[...] = (jnp.where(mask1, p1, 0.0)
                         + jnp.where(mask2, p2, 0.0)) / denom

    lane = lax.broadcasted_iota(jnp.int32, (MC, E), 1)
    eq = lane == jnp.minimum(e, E - 1)
    wloc = wbf_ref[par]
    for mc in range(T // MC):
        rows = pl.ds(mc * MC, MC)
        xb = xb_ref[rows, :]
        gu = lax.dot_general(xb, wloc, (((1,), (0,)), ((), ())),
                             preferred_element_type=jnp.float32)
        g = gu[:, :I]
        u = gu[:, I:]
        a = (0.5 * g) * (1.0 + jnp.tanh(0.5 * g)) * u
        comb_col = jnp.max(jnp.where(eq, comb_ref[rows, :], -1e30), axis=1,
                           keepdims=True)
        scale = jnp.where(e == E, jnp.float32(1.0), comb_col)
        act_ref[rows, :] = (a * scale).astype(jnp.bfloat16)


def _down_body(act_ref, wdall_ref, o_ref):
    o_ref[...] = lax.dot_general(
        act_ref[...], wdall_ref[...], (((1,), (0,)), ((), ())),
        preferred_element_type=jnp.float32)


@jax.jit
def kernel(hidden_states, router_weight, w_gate_up, w_down, shared_gate_up,
           shared_down):
    rwt = router_weight.T.astype(jnp.float32)

    acts, wdall = pl.pallas_call(
        _act_body,
        grid=(NSL,),
        in_specs=[
            pl.BlockSpec((T, H), lambda e: (0, 0)),
            pl.BlockSpec((H, E), lambda e: (0, 0)),
            pl.BlockSpec((1, H, 2 * I), lambda e: (0, 0, 0)),
            pl.BlockSpec((1, H, 2 * I),
                         lambda e: (jnp.minimum(e + 1, E - 1), 0, 0)),
            pl.BlockSpec((H, 2 * I), lambda e: (0, 0)),
            pl.BlockSpec((1, I, H), lambda e: (jnp.minimum(e, E - 1), 0, 0)),
            pl.BlockSpec((I, H), lambda e: (0, 0)),
        ],
        out_specs=[
            pl.BlockSpec((T, I), lambda e: (0, e)),
            pl.BlockSpec((I, H), lambda e: (e, 0)),
        ],
        out_shape=[
            jax.ShapeDtypeStruct((T, NSL * I), jnp.bfloat16),
            jax.ShapeDtypeStruct((NSL * I, H), jnp.bfloat16),
        ],
        scratch_shapes=[
            pltpu.VMEM((2, H, 2 * I), jnp.bfloat16),
            pltpu.VMEM((T, H), jnp.bfloat16),
            pltpu.VMEM((T, E), jnp.float32),
        ],
    )(hidden_states, rwt, w_gate_up, w_gate_up, shared_gate_up, w_down,
      shared_down)

    out = pl.pallas_call(
        _down_body,
        grid=(NI,),
        in_specs=[
            pl.BlockSpec((BM, NSL * I), lambda i: (i, 0)),
            pl.BlockSpec((NSL * I, H), lambda i: (0, 0)),
        ],
        out_specs=pl.BlockSpec((BM, H), lambda i: (i, 0)),
        out_shape=jax.ShapeDtypeStruct((T, H), jnp.float32),
    )(acts, wdall)
    return out


# speedup vs baseline: 1.0203x; 1.0203x over previous
import functools

import jax
import jax.numpy as jnp
from jax import lax
from jax.experimental import pallas as pl
from jax.experimental.pallas import tpu as pltpu

T = 2048
H = 1024
E = 8
K = 2
I = 512
BM = 512
MC = 512
NSL = E + 1
NI = T // BM


def _act_body(x_ref, rwt_ref, wgu_ref, wgun_ref, sgu_ref, wdslice_ref,
              sd_ref, act_ref, wdbf_ref, wbf_ref, xb_ref, comb_ref):
    e = pl.program_id(0)
    par = jnp.remainder(e, 2)
    nxt = jnp.remainder(e + 1, 2)

    @pl.when(e == 0)
    def _():
        wbf_ref[0] = wgu_ref[0].astype(jnp.bfloat16)

    @pl.when(e < E - 1)
    def _():
        wbf_ref[nxt] = wgun_ref[0].astype(jnp.bfloat16)

    @pl.when(e == E - 1)
    def _():
        wbf_ref[nxt] = sgu_ref[...].astype(jnp.bfloat16)

    @pl.when(e < E)
    def _():
        wdbf_ref[...] = wdslice_ref[0].astype(jnp.bfloat16)

    @pl.when(e == E)
    def _():
        wdbf_ref[...] = sd_ref[...].astype(jnp.bfloat16)

    @pl.when(e == 0)
    def _():
        x = x_ref[...]
        xb_ref[...] = x.astype(jnp.bfloat16)
        logits = lax.dot_general(
            x, rwt_ref[...], (((1,), (0,)), ((), ())),
            precision=lax.Precision.DEFAULT,
            preferred_element_type=jnp.float32)
        m = jnp.max(logits, axis=-1, keepdims=True)
        ex = jnp.exp(logits - m)
        probs = ex / jnp.sum(ex, axis=-1, keepdims=True)
        lane = lax.broadcasted_iota(jnp.int32, probs.shape, 1)
        p1 = jnp.max(probs, axis=-1, keepdims=True)
        i1 = jnp.min(jnp.where(probs == p1, lane, E), axis=-1, keepdims=True)
        mask1 = lane == i1
        rest = jnp.where(mask1, -jnp.inf, probs)
        p2 = jnp.max(rest, axis=-1, keepdims=True)
        i2 = jnp.min(jnp.where(rest == p2, lane, E), axis=-1, keepdims=True)
        mask2 = lane == i2
        denom = p1 + p2
        comb_ref[...] = (jnp.where(mask1, p1, 0.0)
                         + jnp.where(mask2, p2, 0.0)) / denom

    lane = lax.broadcasted_iota(jnp.int32, (MC, E), 1)
    eq = lane == jnp.minimum(e, E - 1)
    wloc = wbf_ref[par]
    for mc in range(T // MC):
        rows = pl.ds(mc * MC, MC)
        xb = xb_ref[rows, :]
        gu = lax.dot_general(xb, wloc, (((1,), (0,)), ((), ())),
                             preferred_element_type=jnp.float32)
        g = gu[:, :I]
        u = gu[:, I:]
        a = (0.5 * g) * (1.0 + jnp.tanh(0.5 * g)) * u
        comb_col = jnp.max(jnp.where(eq, comb_ref[rows, :], -1e30), axis=1,
                           keepdims=True)
        scale = jnp.where(e == E, jnp.float32(1.0), comb_col)
        act_ref[rows, :] = (a * scale).astype(jnp.bfloat16)


def _down_body(act_ref, wdall_ref, o_ref):
    o_ref[...] = lax.dot_general(
        act_ref[...], wdall_ref[...], (((1,), (0,)), ((), ())),
        preferred_element_type=jnp.float32)


@jax.jit
def kernel(hidden_states, router_weight, w_gate_up, w_down, shared_gate_up,
           shared_down):
    rwt = router_weight.T.astype(jnp.float32)

    acts, wdall = pl.pallas_call(
        _act_body,
        grid=(NSL,),
        in_specs=[
            pl.BlockSpec((T, H), lambda e: (0, 0)),
            pl.BlockSpec((H, E), lambda e: (0, 0)),
            pl.BlockSpec((1, H, 2 * I), lambda e: (0, 0, 0)),
            pl.BlockSpec((1, H, 2 * I),
                         lambda e: (jnp.minimum(e + 1, E - 1), 0, 0)),
            pl.BlockSpec((H, 2 * I), lambda e: (0, 0)),
            pl.BlockSpec((1, I, H), lambda e: (jnp.minimum(e, E - 1), 0, 0)),
            pl.BlockSpec((I, H), lambda e: (0, 0)),
        ],
        out_specs=[
            pl.BlockSpec((T, I), lambda e: (0, e)),
            pl.BlockSpec((I, H), lambda e: (e, 0)),
        ],
        out_shape=[
            jax.ShapeDtypeStruct((T, NSL * I), jnp.bfloat16),
            jax.ShapeDtypeStruct((NSL * I, H), jnp.bfloat16),
        ],
        scratch_shapes=[
            pltpu.VMEM((2, H, 2 * I), jnp.bfloat16),
            pltpu.VMEM((T, H), jnp.bfloat16),
            pltpu.VMEM((T, E), jnp.float32),
        ],
    )(hidden_states, rwt, w_gate_up, w_gate_up, shared_gate_up, w_down,
      shared_down)

    out = pl.pallas_call(
        _down_body,
        grid=(NI,),
        in_specs=[
            pl.BlockSpec((BM, NSL * I), lambda i: (i, 0)),
            pl.BlockSpec((NSL * I, H), lambda i: (0, 0)),
        ],
        out_specs=pl.BlockSpec((BM, H), lambda i: (i, 0)),
        out_shape=jax.ShapeDtypeStruct((T, H), jnp.float32),
    )(acts, wdall)
    return out


# speedup vs baseline: 1.0416x; 1.0208x over previous
import functools

import jax
import jax.numpy as jnp
from jax import lax
from jax.experimental import pallas as pl
from jax.experimental.pallas import tpu as pltpu

T = 2048
H = 1024
E = 8
K = 2
I = 512
BM = 512
MC = 1024
NSL = E + 1
NI = T // BM


def _act_body(x_ref, rwt_ref, wgu_ref, sgu_ref, wdslice_ref,
              sd_ref, act_ref, wdbf_ref, wbf_ref, xb_ref, comb_ref):
    e = pl.program_id(0)

    @pl.when(e < E)
    def _():
        wbf_ref[...] = wgu_ref[0].astype(jnp.bfloat16)
        wdbf_ref[...] = wdslice_ref[0].astype(jnp.bfloat16)

    @pl.when(e == E)
    def _():
        wbf_ref[...] = sgu_ref[...].astype(jnp.bfloat16)
        wdbf_ref[...] = sd_ref[...].astype(jnp.bfloat16)

    @pl.when(e == 0)
    def _():
        x = x_ref[...]
        xb_ref[...] = x.astype(jnp.bfloat16)
        logits = lax.dot_general(
            x, rwt_ref[...], (((1,), (0,)), ((), ())),
            precision=lax.Precision.DEFAULT,
            preferred_element_type=jnp.float32)
        m = jnp.max(logits, axis=-1, keepdims=True)
        ex = jnp.exp(logits - m)
        probs = ex / jnp.sum(ex, axis=-1, keepdims=True)
        lane = lax.broadcasted_iota(jnp.int32, probs.shape, 1)
        p1 = jnp.max(probs, axis=-1, keepdims=True)
        i1 = jnp.min(jnp.where(probs == p1, lane, E), axis=-1, keepdims=True)
        mask1 = lane == i1
        rest = jnp.where(mask1, -jnp.inf, probs)
        p2 = jnp.max(rest, axis=-1, keepdims=True)
        i2 = jnp.min(jnp.where(rest == p2, lane, E), axis=-1, keepdims=True)
        mask2 = lane == i2
        denom = p1 + p2
        comb_ref[...] = (jnp.where(mask1, p1, 0.0)
                         + jnp.where(mask2, p2, 0.0)) / denom

    lane = lax.broadcasted_iota(jnp.int32, (MC, E), 1)
    eq = lane == jnp.minimum(e, E - 1)
    for mc in range(T // MC):
        rows = pl.ds(mc * MC, MC)
        xb = xb_ref[rows, :]
        gu = lax.dot_general(xb, wbf_ref[...], (((1,), (0,)), ((), ())),
                             preferred_element_type=jnp.float32)
        g = gu[:, :I]
        u = gu[:, I:]
        a = (0.5 * g) * (1.0 + jnp.tanh(0.5 * g)) * u
        comb_col = jnp.max(jnp.where(eq, comb_ref[rows, :], -1e30), axis=1,
                           keepdims=True)
        scale = jnp.where(e == E, jnp.float32(1.0), comb_col)
        act_ref[rows, :] = (a * scale).astype(jnp.bfloat16)


def _down_body(act_ref, wdall_ref, o_ref):
    o_ref[...] = lax.dot_general(
        act_ref[...], wdall_ref[...], (((1,), (0,)), ((), ())),
        preferred_element_type=jnp.float32)


@jax.jit
def kernel(hidden_states, router_weight, w_gate_up, w_down, shared_gate_up,
           shared_down):
    rwt = router_weight.T.astype(jnp.float32)

    acts, wdall = pl.pallas_call(
        _act_body,
        grid=(NSL,),
        in_specs=[
            pl.BlockSpec((T, H), lambda e: (0, 0)),
            pl.BlockSpec((H, E), lambda e: (0, 0)),
            pl.BlockSpec((1, H, 2 * I), lambda e: (jnp.minimum(e, E - 1), 0, 0)),
            pl.BlockSpec((H, 2 * I), lambda e: (0, 0)),
            pl.BlockSpec((1, I, H), lambda e: (jnp.minimum(e, E - 1), 0, 0)),
            pl.BlockSpec((I, H), lambda e: (0, 0)),
        ],
        out_specs=[
            pl.BlockSpec((T, I), lambda e: (0, e)),
            pl.BlockSpec((I, H), lambda e: (e, 0)),
        ],
        out_shape=[
            jax.ShapeDtypeStruct((T, NSL * I), jnp.bfloat16),
            jax.ShapeDtypeStruct((NSL * I, H), jnp.bfloat16),
        ],
        scratch_shapes=[
            pltpu.VMEM((H, 2 * I), jnp.bfloat16),
            pltpu.VMEM((T, H), jnp.bfloat16),
            pltpu.VMEM((T, E), jnp.float32),
        ],
    )(hidden_states, rwt, w_gate_up, shared_gate_up, w_down, shared_down)

    out = pl.pallas_call(
        _down_body,
        grid=(NI,),
        in_specs=[
            pl.BlockSpec((BM, NSL * I), lambda i: (i, 0)),
            pl.BlockSpec((NSL * I, H), lambda i: (0, 0)),
        ],
        out_specs=pl.BlockSpec((BM, H), lambda i: (i, 0)),
        out_shape=jax.ShapeDtypeStruct((T, H), jnp.float32),
    )(acts, wdall)
    return out
